# Initial kernel scaffold; baseline (speedup 1.0000x reference)
import jax
import jax.numpy as jnp
from jax.experimental import pallas as pl


def kernel(S_ad, T_ad, S_eye, T_eye, S_ou, T_ou, bias1, bias2, pooling_weight, pooling_weight_last, pooling_weight1, pooling_weight1_last):
    raise NotImplementedError("write your pallas kernel here")


# speedup vs baseline: 1.3932x; 1.3932x over previous
import jax
import jax.numpy as jnp
from jax.experimental import pallas as pl
from jax.experimental.pallas import tpu as pltpu

N = 4096
D = 64
NPASS = 78
_TRI = [t * (t + 1) // 2 for t in range(13)]


def _mm_kernel(aS_ref, xS_ref, aT_ref, xT_ref, oS_ref, oT_ref):
    dn = (((1,), (0,)), ((), ()))
    oS_ref[...] = jax.lax.dot_general(
        aS_ref[...], xS_ref[...], dn,
        precision=jax.lax.Precision.HIGHEST,
        preferred_element_type=jnp.float32)
    oT_ref[...] = jax.lax.dot_general(
        aT_ref[...], xT_ref[...], dn,
        precision=jax.lax.Precision.HIGHEST,
        preferred_element_type=jnp.float32)


def _mm_call(A_S, X_S, A_T, X_T):
    blk = 256
    grid = (N // blk,)
    return pl.pallas_call(
        _mm_kernel,
        grid=grid,
        in_specs=[
            pl.BlockSpec((blk, N), lambda i: (i, 0)),
            pl.BlockSpec((N, D), lambda i: (0, 0)),
            pl.BlockSpec((blk, N), lambda i: (i, 0)),
            pl.BlockSpec((N, D), lambda i: (0, 0)),
        ],
        out_specs=[
            pl.BlockSpec((blk, D), lambda i: (i, 0)),
            pl.BlockSpec((blk, D), lambda i: (i, 0)),
        ],
        out_shape=[
            jax.ShapeDtypeStruct((N, D), jnp.float32),
            jax.ShapeDtypeStruct((N, D), jnp.float32),
        ],
    )(A_S, X_S, A_T, X_T)


def _bitonic_pass(x, j, k, m):
    bit_j = (m & j) != 0
    asc = (m & k) == 0
    partner = jnp.where(bit_j, pltpu.roll(x, j, 1), pltpu.roll(x, N - j, 1))
    return jnp.where(bit_j == asc, jnp.maximum(x, partner),
                     jnp.minimum(x, partner))


def _onehot_write(acc, idx, val):
    lane = jax.lax.broadcasted_iota(jnp.int32, (1, 128), 1)
    return acc + jnp.where(lane == idx, val, 0.0)


def _sortloss_kernel(x_ref, y_ref, p_ref, pl_ref, out_ref, sS_ref, sT_ref):
    p = pl.program_id(0)

    @pl.when(p == 0)
    def _init():
        sS_ref[...] = x_ref[...]
        sT_ref[...] = y_ref[...]

    n = jnp.int32(0)
    for t in range(1, 13):
        n = n + jnp.where(p >= _TRI[t - 1], 1, 0).astype(jnp.int32)
    j_exp = (n * (n + 1) // 2 - 1 - p).astype(jnp.int32)
    j = (jnp.int32(1) << j_exp)
    k = (jnp.int32(1) << n)

    m = jax.lax.broadcasted_iota(jnp.int32, (D, N), 1)
    sS_ref[...] = _bitonic_pass(sS_ref[...], j, k, m)
    sT_ref[...] = _bitonic_pass(sT_ref[...], j, k, m)

    @pl.when(p == NPASS - 1)
    def _losses():
        sS = sS_ref[...]
        sT = sT_ref[...]
        maxS = sS[:, N - 1:N]
        maxT = sT[:, N - 1:N]
        minS = sS[:, 0:1]
        minT = sT[:, 0:1]
        meanS = jnp.sum(sS, axis=1, keepdims=True) * (1.0 / N)
        meanT = jnp.sum(sT, axis=1, keepdims=True) * (1.0 / N)
        Pma = jnp.sum((jnp.maximum(maxS, 0.0) - jnp.maximum(maxT, 0.0)) ** 2)
        Pmi = jnp.sum((jnp.minimum(minS, 0.0) - jnp.minimum(minT, 0.0)) ** 2)
        Pav = jnp.sum((meanS - meanT) ** 2)
        top_chunks = []
        bot_chunks = []
        for i in range(8):
            hi = N - 64 * i
            cS = jnp.sum(jnp.maximum(sS[:, hi - 64:hi], 0.0), axis=1,
                         keepdims=True)
            cT = jnp.sum(jnp.maximum(sT[:, hi - 64:hi], 0.0), axis=1,
                         keepdims=True)
            top_chunks.append((cS - cT) * (1.0 / 64.0))
            bS = jnp.sum(jnp.minimum(sS[:, 64 * i:64 * i + 64], 0.0), axis=1,
                         keepdims=True)
            bT = jnp.sum(jnp.minimum(sT[:, 64 * i:64 * i + 64], 0.0), axis=1,
                         keepdims=True)
            bot_chunks.append((bS - bT) * (1.0 / 64.0))
        Mtop = jnp.concatenate(top_chunks, axis=1)
        Mbot = jnp.concatenate(bot_chunks, axis=1)
        dn = (((1,), (0,)), ((), ()))
        ra1 = jnp.sum(jax.lax.dot_general(
            Mtop, p_ref[...], dn,
            precision=jax.lax.Precision.HIGHEST,
            preferred_element_type=jnp.float32) ** 2)
        ra2 = jnp.sum(jax.lax.dot_general(
            Mbot, pl_ref[...], dn,
            precision=jax.lax.Precision.HIGHEST,
            preferred_element_type=jnp.float32) ** 2)
        acc = jnp.zeros((1, 128), jnp.float32)
        acc = _onehot_write(acc, 0, Pma)
        acc = _onehot_write(acc, 1, Pmi)
        acc = _onehot_write(acc, 2, Pav)
        acc = _onehot_write(acc, 3, ra1)
        acc = _onehot_write(acc, 4, ra2)
        out_ref[...] = acc


def _sortloss_call(Xt, Yt, P, PL):
    return pl.pallas_call(
        _sortloss_kernel,
        grid=(NPASS,),
        in_specs=[
            pl.BlockSpec((D, N), lambda p: (0, 0)),
            pl.BlockSpec((D, N), lambda p: (0, 0)),
            pl.BlockSpec((8, 16), lambda p: (0, 0)),
            pl.BlockSpec((8, 16), lambda p: (0, 0)),
        ],
        out_specs=pl.BlockSpec((1, 128), lambda p: (0, 0)),
        out_shape=jax.ShapeDtypeStruct((1, 128), jnp.float32),
        scratch_shapes=[
            pltpu.VMEM((D, N), jnp.float32),
            pltpu.VMEM((D, N), jnp.float32),
        ],
    )(Xt, Yt, P, PL)


def kernel(S_ad, T_ad, S_eye, T_eye, S_ou, T_ou, bias1, bias2, pooling_weight,
           pooling_weight_last, pooling_weight1, pooling_weight1_last):
    XS = S_ou[0]
    XT = T_ou[0]
    losses = []
    for i in range(2):
        XS, XT = _mm_call(S_ad, XS, T_ad, XT)
        Xt = (XS - bias1).T
        Yt = (XT - bias2).T
        P = pooling_weight if i == 0 else pooling_weight1
        PL = pooling_weight_last if i == 0 else pooling_weight1_last
        out = _sortloss_call(Xt, Yt, P, PL)
        losses.append(out[0, :5])
    return jnp.stack(losses, axis=1)


# speedup vs baseline: 1.4193x; 1.0188x over previous
import jax
import jax.numpy as jnp
from jax.experimental import pallas as pl
from jax.experimental.pallas import tpu as pltpu

N = 4096
D = 64
NPASS = 78
_TRI = [t * (t + 1) // 2 for t in range(13)]


def _mm_kernel(aS_ref, xS_ref, aT_ref, xT_ref, oS_ref, oT_ref):
    dn = (((1,), (0,)), ((), ()))
    oS_ref[...] = jax.lax.dot_general(
        aS_ref[...], xS_ref[...], dn,
        precision=jax.lax.Precision.HIGHEST,
        preferred_element_type=jnp.float32)
    oT_ref[...] = jax.lax.dot_general(
        aT_ref[...], xT_ref[...], dn,
        precision=jax.lax.Precision.HIGHEST,
        preferred_element_type=jnp.float32)


def _mm_call(A_S, X_S, A_T, X_T):
    blk = 256
    grid = (N // blk,)
    return pl.pallas_call(
        _mm_kernel,
        grid=grid,
        in_specs=[
            pl.BlockSpec((blk, N), lambda i: (i, 0)),
            pl.BlockSpec((N, D), lambda i: (0, 0)),
            pl.BlockSpec((blk, N), lambda i: (i, 0)),
            pl.BlockSpec((N, D), lambda i: (0, 0)),
        ],
        out_specs=[
            pl.BlockSpec((blk, D), lambda i: (i, 0)),
            pl.BlockSpec((blk, D), lambda i: (i, 0)),
        ],
        out_shape=[
            jax.ShapeDtypeStruct((N, D), jnp.float32),
            jax.ShapeDtypeStruct((N, D), jnp.float32),
        ],
    )(A_S, X_S, A_T, X_T)


def _onehot_write(acc, idx, val):
    lane = jax.lax.broadcasted_iota(jnp.int32, (1, 128), 1)
    return acc + jnp.where(lane == idx, val, 0.0)


def _local_pass(ref, j, k, t):
    x = ref[t]
    lane = jax.lax.broadcasted_iota(jnp.int32, (D, 128), 1)
    gidx = lane + t * 128
    bit_j = (lane & j) != 0
    asc = (gidx & k) == 0
    partner = jnp.where(bit_j, pltpu.roll(x, j, 1), pltpu.roll(x, 128 - j, 1))
    ref[t] = jnp.where(bit_j == asc, jnp.maximum(x, partner),
                       jnp.minimum(x, partner))


def _cross_pass(ref, jt, k, q):
    t_lo = (q & (jt - 1)) | ((q & ~(jt - 1)) << 1)
    t_hi = t_lo | jt
    a = ref[t_lo]
    b = ref[t_hi]
    asc = ((t_lo * 128) & k) == 0
    mn = jnp.minimum(a, b)
    mx = jnp.maximum(a, b)
    ref[t_lo] = jnp.where(asc, mn, mx)
    ref[t_hi] = jnp.where(asc, mx, mn)


def _sortloss_kernel(x_ref, y_ref, p_ref, pl_ref, out_ref, sS_ref, sT_ref):
    p = pl.program_id(0)

    @pl.when(p == 0)
    def _init():
        for t in range(32):
            sS_ref[t] = x_ref[:, 128 * t:128 * (t + 1)]
            sT_ref[t] = y_ref[:, 128 * t:128 * (t + 1)]

    n = jnp.int32(0)
    for t in range(1, 13):
        n = n + jnp.where(p >= _TRI[t - 1], 1, 0).astype(jnp.int32)
    j_exp = (n * (n + 1) // 2 - 1 - p).astype(jnp.int32)
    j = (jnp.int32(1) << j_exp)
    k = (jnp.int32(1) << n)
    jt = j >> 7

    @pl.when(j < 128)
    def _local():
        def body(t, _):
            _local_pass(sS_ref, j, k, t)
            _local_pass(sT_ref, j, k, t)
            return _
        jax.lax.fori_loop(0, 32, body, 0)

    @pl.when(j >= 128)
    def _cross():
        def body(q, _):
            _cross_pass(sS_ref, jt, k, q)
            _cross_pass(sT_ref, jt, k, q)
            return _
        jax.lax.fori_loop(0, 16, body, 0)

    @pl.when(p == NPASS - 1)
    def _losses():
        maxS = sS_ref[31][:, 127:128]
        maxT = sT_ref[31][:, 127:128]
        minS = sS_ref[0][:, 0:1]
        minT = sT_ref[0][:, 0:1]
        accS = sS_ref[0]
        accT = sT_ref[0]
        for t in range(1, 32):
            accS = accS + sS_ref[t]
            accT = accT + sT_ref[t]
        meanS = jnp.sum(accS, axis=1, keepdims=True) * (1.0 / N)
        meanT = jnp.sum(accT, axis=1, keepdims=True) * (1.0 / N)
        Pma = jnp.sum((jnp.maximum(maxS, 0.0) - jnp.maximum(maxT, 0.0)) ** 2)
        Pmi = jnp.sum((jnp.minimum(minS, 0.0) - jnp.minimum(minT, 0.0)) ** 2)
        Pav = jnp.sum((meanS - meanT) ** 2)
        top_chunks = []
        bot_chunks = []
        for i in range(8):
            lo = N - 64 * (i + 1)
            tt, ll = lo // 128, lo % 128
            cS = jnp.sum(jnp.maximum(sS_ref[tt][:, ll:ll + 64], 0.0), axis=1,
                         keepdims=True)
            cT = jnp.sum(jnp.maximum(sT_ref[tt][:, ll:ll + 64], 0.0), axis=1,
                         keepdims=True)
            top_chunks.append((cS - cT) * (1.0 / 64.0))
            lob = 64 * i
            tb, lb = lob // 128, lob % 128
            bS = jnp.sum(jnp.minimum(sS_ref[tb][:, lb:lb + 64], 0.0), axis=1,
                         keepdims=True)
            bT = jnp.sum(jnp.minimum(sT_ref[tb][:, lb:lb + 64], 0.0), axis=1,
                         keepdims=True)
            bot_chunks.append((bS - bT) * (1.0 / 64.0))
        Mtop = jnp.concatenate(top_chunks, axis=1)
        Mbot = jnp.concatenate(bot_chunks, axis=1)
        dn = (((1,), (0,)), ((), ()))
        ra1 = jnp.sum(jax.lax.dot_general(
            Mtop, p_ref[...], dn,
            precision=jax.lax.Precision.HIGHEST,
            preferred_element_type=jnp.float32) ** 2)
        ra2 = jnp.sum(jax.lax.dot_general(
            Mbot, pl_ref[...], dn,
            precision=jax.lax.Precision.HIGHEST,
            preferred_element_type=jnp.float32) ** 2)
        acc = jnp.zeros((1, 128), jnp.float32)
        acc = _onehot_write(acc, 0, Pma)
        acc = _onehot_write(acc, 1, Pmi)
        acc = _onehot_write(acc, 2, Pav)
        acc = _onehot_write(acc, 3, ra1)
        acc = _onehot_write(acc, 4, ra2)
        out_ref[...] = acc


def _sortloss_call(Xt, Yt, P, PL):
    return pl.pallas_call(
        _sortloss_kernel,
        grid=(NPASS,),
        in_specs=[
            pl.BlockSpec((D, N), lambda p: (0, 0)),
            pl.BlockSpec((D, N), lambda p: (0, 0)),
            pl.BlockSpec((8, 16), lambda p: (0, 0)),
            pl.BlockSpec((8, 16), lambda p: (0, 0)),
        ],
        out_specs=pl.BlockSpec((1, 128), lambda p: (0, 0)),
        out_shape=jax.ShapeDtypeStruct((1, 128), jnp.float32),
        scratch_shapes=[
            pltpu.VMEM((32, D, 128), jnp.float32),
            pltpu.VMEM((32, D, 128), jnp.float32),
        ],
    )(Xt, Yt, P, PL)


def kernel(S_ad, T_ad, S_eye, T_eye, S_ou, T_ou, bias1, bias2, pooling_weight,
           pooling_weight_last, pooling_weight1, pooling_weight1_last):
    XS = S_ou[0]
    XT = T_ou[0]
    losses = []
    for i in range(2):
        XS, XT = _mm_call(S_ad, XS, T_ad, XT)
        Xt = (XS - bias1).T
        Yt = (XT - bias2).T
        P = pooling_weight if i == 0 else pooling_weight1
        PL = pooling_weight_last if i == 0 else pooling_weight1_last
        out = _sortloss_call(Xt, Yt, P, PL)
        losses.append(out[0, :5])
    return jnp.stack(losses, axis=1)
